# baseline (speedup 1.0000x reference)
import functools
import math

import jax
import jax.numpy as jnp
from jax.experimental import pallas as pl
from jax.experimental.pallas import tpu as pltpu

_LANE = 128


def _adain_kernel(x_ref, w_ref, b_ref, o_ref, *, eps, inv_n):
    x = x_ref[...].astype(jnp.float32)
    s1 = jnp.sum(x, axis=-1, keepdims=True)
    s2 = jnp.sum(x * x, axis=-1, keepdims=True)
    mean = s1 * inv_n
    var = jnp.maximum(s2 * inv_n - mean * mean, 0.0)
    scale = jax.lax.rsqrt(var + eps) * w_ref[...].astype(jnp.float32)
    shift = b_ref[...].astype(jnp.float32) - mean * scale
    o_ref[...] = (x * scale + shift).astype(o_ref.dtype)


def _pick_row_tile(bc, s, itemsize):
    budget = 24 * 1024 * 1024
    bytes_per_row = s * itemsize * 4
    min_rows = max(8, 32 // itemsize)
    tr = max(min_rows, (budget // max(bytes_per_row, 1)) // min_rows * min_rows)
    tr = min(tr, bc)
    cand = tr
    while cand > min_rows and not (bc % cand == 0 and (bc // cand) % 2 == 0
                                   and bc // cand >= 4):
        cand -= min_rows
    if cand >= min_rows and bc % cand == 0 and (bc // cand) % 2 == 0:
        return cand
    return tr


def adain_forward(x, weight, bias, *, eps=1e-5):
    orig_shape = x.shape
    B, C = orig_shape[0], orig_shape[1]
    S = math.prod(orig_shape[2:])
    BC = B * C
    dtype = x.dtype
    itemsize = jnp.dtype(dtype).itemsize

    x2d = x.reshape(BC, S)
    w2d = weight.reshape(BC, 1)
    b2d = bias.reshape(BC, 1)

    tr = _pick_row_tile(BC, S, itemsize)
    n_blocks = pl.cdiv(BC, tr)
    bc_pad = n_blocks * tr
    if bc_pad != BC:
        w2d = jnp.pad(w2d, ((0, bc_pad - BC), (0, 0)))
        b2d = jnp.pad(b2d, ((0, bc_pad - BC), (0, 0)))

    out2d = pl.pallas_call(
        functools.partial(_adain_kernel, eps=eps, inv_n=1.0 / S),
        out_shape=jax.ShapeDtypeStruct((BC, S), dtype),
        grid=(n_blocks,),
        in_specs=[
            pl.BlockSpec((tr, S), lambda i: (i, 0)),
            pl.BlockSpec((tr, 1), lambda i: (i, 0)),
            pl.BlockSpec((tr, 1), lambda i: (i, 0)),
        ],
        out_specs=pl.BlockSpec((tr, S), lambda i: (i, 0)),
        compiler_params=pltpu.CompilerParams(
            dimension_semantics=("parallel",),
            vmem_limit_bytes=56 * 1024 * 1024),
    )(x2d, w2d, b2d)

    return out2d.reshape(orig_shape)


def kernel(x, weight, bias):
    return adain_forward(x, weight, bias, eps=1e-5)


# speedup vs baseline: 1.6244x; 1.6244x over previous
import functools
import math

import jax
import jax.numpy as jnp
from jax.experimental import pallas as pl
from jax.experimental.pallas import tpu as pltpu


def _adain_kernel(x_ref, w_ref, b_ref, o_ref, *, eps, inv_n):
    x = x_ref[...].astype(jnp.float32)
    s1 = jnp.sum(x, axis=(1, 2), keepdims=True)
    s2 = jnp.sum(x * x, axis=(1, 2), keepdims=True)
    mean = s1 * inv_n
    var = jnp.maximum(s2 * inv_n - mean * mean, 0.0)
    w = w_ref[...].astype(jnp.float32).reshape(mean.shape)
    b = b_ref[...].astype(jnp.float32).reshape(mean.shape)
    scale = jax.lax.rsqrt(var + eps) * w
    shift = b - mean * scale
    o_ref[...] = (x * scale + shift).astype(o_ref.dtype)


def _pick_row_tile(bc, hw_bytes):
    budget = 20 * 1024 * 1024
    bytes_per_row = hw_bytes * 6
    min_rows = 8
    tr = max(min_rows, (budget // max(bytes_per_row, 1)) // min_rows * min_rows)
    tr = min(tr, bc)
    cand = tr
    while cand > min_rows and not (bc % cand == 0 and (bc // cand) % 2 == 0
                                   and bc // cand >= 4):
        cand -= min_rows
    if cand >= min_rows and bc % cand == 0 and (bc // cand) % 2 == 0:
        return cand
    return tr


def adain_forward(x, weight, bias, *, eps=1e-5):
    orig_shape = x.shape
    B, C = orig_shape[0], orig_shape[1]
    H = orig_shape[2] if len(orig_shape) > 2 else 1
    W = math.prod(orig_shape[3:]) if len(orig_shape) > 3 else 1
    S = H * W
    BC = B * C
    dtype = x.dtype
    itemsize = jnp.dtype(dtype).itemsize

    x3d = x.reshape(BC, H, W)
    w2d = weight.reshape(BC, 1)
    b2d = bias.reshape(BC, 1)

    w_pad = max(W, 128) if W < 128 else W
    tr = _pick_row_tile(BC, H * w_pad * itemsize)
    n_blocks = pl.cdiv(BC, tr)
    bc_pad = n_blocks * tr
    if bc_pad != BC:
        w2d = jnp.pad(w2d, ((0, bc_pad - BC), (0, 0)))
        b2d = jnp.pad(b2d, ((0, bc_pad - BC), (0, 0)))

    out3d = pl.pallas_call(
        functools.partial(_adain_kernel, eps=eps, inv_n=1.0 / S),
        out_shape=jax.ShapeDtypeStruct((BC, H, W), dtype),
        grid=(n_blocks,),
        in_specs=[
            pl.BlockSpec((tr, H, W), lambda i: (i, 0, 0)),
            pl.BlockSpec((tr, 1), lambda i: (i, 0)),
            pl.BlockSpec((tr, 1), lambda i: (i, 0)),
        ],
        out_specs=pl.BlockSpec((tr, H, W), lambda i: (i, 0, 0)),
        compiler_params=pltpu.CompilerParams(
            dimension_semantics=("parallel",),
            vmem_limit_bytes=56 * 1024 * 1024),
    )(x3d, w2d, b2d)

    return out3d.reshape(orig_shape)


def kernel(x, weight, bias):
    return adain_forward(x, weight, bias, eps=1e-5)
